# baseline (speedup 1.0000x reference)
import jax
import jax.numpy as jnp
from jax import lax
from jax.experimental import pallas as pl
from jax.experimental.pallas import tpu as pltpu
from jax.experimental.pallas import tpu_sc as plsc

BATCH = 16384
IN_W = 1024
OUT_W = 512

NC = 2
NS = 16
NW = NC * NS
LANES = 16

ROWS_PER_W = BATCH // NW
RB = 32
NBATCH = ROWS_PER_W // RB
JGROUPS = OUT_W // LANES
U = 8
CHUNK = 128
SR = 128
NG = ROWS_PER_W // SR
BPG = SR // RB


def _gather_body(x_hbm, gates_hbm, out_hbm, idx_v,
                 st_a, st_b, out_a, out_b, out_c,
                 in_sem_a, in_sem_b, out_sem_a, out_sem_b, out_sem_c):
    wid = lax.axis_index("s") * NC + lax.axis_index("c")
    row0 = wid * ROWS_PER_W

    pltpu.sync_copy(gates_hbm, idx_v)

    def red_body(k, carry):
        lo, hi = carry
        v = idx_v[pl.ds(k * LANES, LANES)]
        return jnp.minimum(lo, v), jnp.maximum(hi, v)

    lo_v, hi_v = lax.fori_loop(
        0, JGROUPS, red_body,
        (jnp.full((LANES,), IN_W - 1, jnp.int32),
         jnp.zeros((LANES,), jnp.int32)))
    c0 = jnp.min(lo_v) // CHUNK
    c1 = jnp.max(hi_v) // CHUNK
    nch = c1 - c0 + 1
    col_base = c0 * CHUNK

    stages = [st_a, st_b]
    outs = [out_a, out_b, out_c]
    in_sems = [in_sem_a, in_sem_b]
    out_sems = [out_sem_a, out_sem_b, out_sem_c]

    def in_desc(g):
        return pltpu.make_async_copy(
            x_hbm.at[pl.ds(row0 + g * SR, SR), pl.ds(col_base, CHUNK)],
            stages[g % 2], in_sems[g % 2])

    def out_desc(b):
        return pltpu.make_async_copy(
            outs[b % 3], out_hbm.at[pl.ds(row0 + b * RB, RB), :],
            out_sems[b % 3])

    def compute(st_ref, roff, ob_ref, iv_base):
        def group_body(j, _):
            iv = idx_v[pl.ds(j * LANES, LANES)] - iv_base
            joff = j * LANES

            @plsc.parallel_loop(0, RB, step=1, unroll=U)
            def _(r):
                rsplat = jnp.zeros((LANES,), jnp.int32) + (roff + r)
                vals = plsc.load_gather(st_ref, [rsplat, iv])
                ob_ref[r, pl.ds(joff, LANES)] = vals.astype(jnp.int32)

            return 0

        lax.fori_loop(0, JGROUPS, group_body, 0, unroll=False)

    @pl.when(nch == 1)
    def _fast():
        in_desc(0).start()
        for g in range(NG):
            if g + 1 < NG:
                in_desc(g + 1).start()
            in_desc(g).wait()
            for k in range(BPG):
                b = g * BPG + k
                if b >= 3:
                    out_desc(b - 3).wait()
                compute(stages[g % 2], k * RB, outs[b % 3], col_base)
                out_desc(b).start()
        out_desc(NBATCH - 3).wait()
        out_desc(NBATCH - 2).wait()
        out_desc(NBATCH - 1).wait()

    @pl.when(nch > 1)
    def _general():
        def batch_body(b, _):
            rbase = row0 + b * RB

            def chunk_body(c, _):
                pltpu.sync_copy(
                    x_hbm.at[pl.ds(rbase, RB),
                             pl.ds(col_base + c * CHUNK, CHUNK)],
                    st_a.at[pl.ds(0, RB), :])
                cb = col_base + c * CHUNK

                def group_body(j, _):
                    lc = idx_v[pl.ds(j * LANES, LANES)] - cb
                    m = (lc >= 0) & (lc < CHUNK)
                    lcc = jnp.clip(lc, 0, CHUNK - 1)
                    joff = j * LANES

                    @plsc.parallel_loop(0, RB, step=1, unroll=U)
                    def _(r):
                        rsplat = jnp.zeros((LANES,), jnp.int32) + r
                        vals = plsc.load_gather(st_a, [rsplat, lcc])
                        prev = out_a[r, pl.ds(joff, LANES)]
                        out_a[r, pl.ds(joff, LANES)] = jnp.where(
                            m, vals.astype(jnp.int32), prev)

                    return 0

                lax.fori_loop(0, JGROUPS, group_body, 0, unroll=False)
                return 0

            lax.fori_loop(0, nch, chunk_body, 0)
            pltpu.sync_copy(out_a, out_hbm.at[pl.ds(rbase, RB), :])
            return 0

        lax.fori_loop(0, NBATCH, batch_body, 0)


@jax.jit
def _gather(x, output_gates):
    mesh = plsc.VectorSubcoreMesh(core_axis_name="c", subcore_axis_name="s")
    return pl.kernel(
        _gather_body,
        out_type=jax.ShapeDtypeStruct((BATCH, OUT_W), jnp.int32),
        mesh=mesh,
        compiler_params=pltpu.CompilerParams(
            needs_layout_passes=False,
            use_tc_tiling_on_sc=True,
        ),
        scratch_types=[
            pltpu.VMEM((OUT_W,), jnp.int32),
            pltpu.VMEM((SR, CHUNK), jnp.float32),
            pltpu.VMEM((SR, CHUNK), jnp.float32),
            pltpu.VMEM((RB, OUT_W), jnp.int32),
            pltpu.VMEM((RB, OUT_W), jnp.int32),
            pltpu.VMEM((RB, OUT_W), jnp.int32),
            pltpu.SemaphoreType.DMA,
            pltpu.SemaphoreType.DMA,
            pltpu.SemaphoreType.DMA,
            pltpu.SemaphoreType.DMA,
            pltpu.SemaphoreType.DMA,
        ],
    )(x, output_gates)


def kernel(x, output_gates):
    return _gather(x, output_gates)


# speedup vs baseline: 1.0954x; 1.0954x over previous
import jax
import jax.numpy as jnp
from jax import lax
from jax.experimental import pallas as pl
from jax.experimental.pallas import tpu as pltpu
from jax.experimental.pallas import tpu_sc as plsc

BATCH = 16384
IN_W = 1024
OUT_W = 512

NC = 2
NS = 16
NW = NC * NS
LANES = 16

ROWS_PER_W = BATCH // NW
RB = 64
NBATCH = ROWS_PER_W // RB
JGROUPS = OUT_W // LANES
U = 8
CHUNK = 128
SR = 128
NG = ROWS_PER_W // SR
BPG = SR // RB


def _gather_body(x_hbm, gates_hbm, out_hbm, idx_v,
                 st_a, st_b, out_a, out_b,
                 in_sem_a, in_sem_b, out_sem_a, out_sem_b):
    wid = lax.axis_index("s") * NC + lax.axis_index("c")
    row0 = wid * ROWS_PER_W

    pltpu.sync_copy(gates_hbm, idx_v)

    def red_body(k, carry):
        lo, hi = carry
        v = idx_v[pl.ds(k * LANES, LANES)]
        return jnp.minimum(lo, v), jnp.maximum(hi, v)

    lo_v, hi_v = lax.fori_loop(
        0, JGROUPS, red_body,
        (jnp.full((LANES,), IN_W - 1, jnp.int32),
         jnp.zeros((LANES,), jnp.int32)))
    c0 = jnp.min(lo_v) // CHUNK
    c1 = jnp.max(hi_v) // CHUNK
    nch = c1 - c0 + 1
    col_base = c0 * CHUNK

    stages = [st_a, st_b]
    outs = [out_a, out_b]
    in_sems = [in_sem_a, in_sem_b]
    out_sems = [out_sem_a, out_sem_b]

    def in_desc(g):
        return pltpu.make_async_copy(
            x_hbm.at[pl.ds(row0 + g * SR, SR), pl.ds(col_base, CHUNK)],
            stages[g % 2], in_sems[g % 2])

    def out_desc(b):
        return pltpu.make_async_copy(
            outs[b % 2], out_hbm.at[pl.ds(row0 + b * RB, RB), :],
            out_sems[b % 2])

    def compute(st_ref, roff, ob_ref, iv_base):
        def group_body(j, _):
            iv = idx_v[pl.ds(j * LANES, LANES)] - iv_base
            joff = j * LANES

            @plsc.parallel_loop(0, RB, step=1, unroll=U)
            def _(r):
                rsplat = jnp.zeros((LANES,), jnp.int32) + (roff + r)
                vals = plsc.load_gather(st_ref, [rsplat, iv])
                ob_ref[r, pl.ds(joff, LANES)] = vals.astype(jnp.int32)

            return 0

        lax.fori_loop(0, JGROUPS, group_body, 0, unroll=False)

    @pl.when(nch == 1)
    def _fast():
        in_desc(0).start()
        for g in range(NG):
            if g + 1 < NG:
                in_desc(g + 1).start()
            in_desc(g).wait()
            for k in range(BPG):
                b = g * BPG + k
                if b >= 2:
                    out_desc(b - 2).wait()
                compute(stages[g % 2], k * RB, outs[b % 2], col_base)
                out_desc(b).start()
        out_desc(NBATCH - 2).wait()
        out_desc(NBATCH - 1).wait()

    @pl.when(nch > 1)
    def _general():
        def batch_body(b, _):
            rbase = row0 + b * RB

            def chunk_body(c, _):
                pltpu.sync_copy(
                    x_hbm.at[pl.ds(rbase, RB),
                             pl.ds(col_base + c * CHUNK, CHUNK)],
                    st_a.at[pl.ds(0, RB), :])
                cb = col_base + c * CHUNK

                def group_body(j, _):
                    lc = idx_v[pl.ds(j * LANES, LANES)] - cb
                    m = (lc >= 0) & (lc < CHUNK)
                    lcc = jnp.clip(lc, 0, CHUNK - 1)
                    joff = j * LANES

                    @plsc.parallel_loop(0, RB, step=1, unroll=U)
                    def _(r):
                        rsplat = jnp.zeros((LANES,), jnp.int32) + r
                        vals = plsc.load_gather(st_a, [rsplat, lcc])
                        prev = out_a[r, pl.ds(joff, LANES)]
                        out_a[r, pl.ds(joff, LANES)] = jnp.where(
                            m, vals.astype(jnp.int32), prev)

                    return 0

                lax.fori_loop(0, JGROUPS, group_body, 0, unroll=False)
                return 0

            lax.fori_loop(0, nch, chunk_body, 0)
            pltpu.sync_copy(out_a, out_hbm.at[pl.ds(rbase, RB), :])
            return 0

        lax.fori_loop(0, NBATCH, batch_body, 0)


@jax.jit
def _gather(x, output_gates):
    mesh = plsc.VectorSubcoreMesh(core_axis_name="c", subcore_axis_name="s")
    return pl.kernel(
        _gather_body,
        out_type=jax.ShapeDtypeStruct((BATCH, OUT_W), jnp.int32),
        mesh=mesh,
        compiler_params=pltpu.CompilerParams(
            needs_layout_passes=False,
            use_tc_tiling_on_sc=True,
        ),
        scratch_types=[
            pltpu.VMEM((OUT_W,), jnp.int32),
            pltpu.VMEM((SR, CHUNK), jnp.float32),
            pltpu.VMEM((SR, CHUNK), jnp.float32),
            pltpu.VMEM((RB, OUT_W), jnp.int32),
            pltpu.VMEM((RB, OUT_W), jnp.int32),
            pltpu.SemaphoreType.DMA,
            pltpu.SemaphoreType.DMA,
            pltpu.SemaphoreType.DMA,
            pltpu.SemaphoreType.DMA,
        ],
    )(x, output_gates)


def kernel(x, output_gates):
    return _gather(x, output_gates)
